# baseline (speedup 1.0000x reference)
import functools

import jax
import jax.numpy as jnp
from jax import lax
from jax.experimental import pallas as pl
from jax.experimental.pallas import tpu as pltpu
from jax.experimental.pallas import tpu_sc as plsc

_K = 8192
_D = 32
_ROWS = 128
_COMMIT = 0.25


def _finalize_body(z_ref, g_ref, counts_ref, q_ref, loss_ref, perp_ref,
                   acc_ref):
    i = pl.program_id(0)
    nsteps = pl.num_programs(0)
    z = z_ref[...]
    g = g_ref[...]
    q_ref[...] = z + (g - z)
    diff = g - z

    @pl.when(i == 0)
    def _():
        acc_ref[0, 0] = 0.0

    acc_ref[0, 0] += jnp.sum(diff * diff)

    @pl.when(i == nsteps - 1)
    def _():
        n_elems = nsteps * _ROWS * _D
        loss_ref[...] = jnp.reshape(
            (1.0 + _COMMIT) * acc_ref[0, 0] / n_elems, (1, 1))
        counts = jnp.sum(counts_ref[...], axis=0, keepdims=True)
        p = counts * (1.0 / (nsteps * _ROWS))
        perp_ref[...] = jnp.reshape(
            jnp.exp(-jnp.sum(p * jnp.log(p + 1e-10))), (1, 1))


def _tc_finalize(flat, gathered, counts):
    n = flat.shape[0]
    return pl.pallas_call(
        _finalize_body,
        grid=(n // _ROWS,),
        in_specs=[
            pl.BlockSpec((_ROWS, _D), lambda i: (i, 0)),
            pl.BlockSpec((_ROWS, _D), lambda i: (i, 0)),
            pl.BlockSpec((32, _K), lambda i: (0, 0)),
        ],
        out_specs=[
            pl.BlockSpec((_ROWS, _D), lambda i: (i, 0)),
            pl.BlockSpec((1, 1), lambda i: (0, 0)),
            pl.BlockSpec((1, 1), lambda i: (0, 0)),
        ],
        out_shape=[
            jax.ShapeDtypeStruct((n, _D), jnp.float32),
            jax.ShapeDtypeStruct((1, 1), jnp.float32),
            jax.ShapeDtypeStruct((1, 1), jnp.float32),
        ],
        scratch_shapes=[
            pltpu.SMEM((1, 1), jnp.float32),
        ],
    )(flat, gathered, counts)


def _sc_gather_hist(table, idx):
    info = plsc.get_sparse_core_info()
    nc, nl = info.num_cores, info.num_lanes
    nw = nc * info.num_subcores
    b, d = idx.shape[0], table.shape[1]
    b_per_w = b // nw
    mesh = plsc.VectorSubcoreMesh(core_axis_name="c", subcore_axis_name="s")

    @functools.partial(
        pl.kernel, mesh=mesh,
        out_type=[
            jax.ShapeDtypeStruct((b, d), jnp.float32),
            jax.ShapeDtypeStruct((nw, _K), jnp.float32),
        ],
        scratch_types=[
            pltpu.VMEM((b_per_w,), jnp.int32),
            pltpu.VMEM((b_per_w, d), jnp.float32),
            pltpu.VMEM((_K,), jnp.float32),
            pltpu.SemaphoreType.DMA,
        ],
        compiler_params=pltpu.CompilerParams(
            use_tc_tiling_on_sc=False, needs_layout_passes=False),
    )
    def k(table_hbm, idx_hbm, out_hbm, cnt_hbm, idx_v, rows_v, cnt_v, sem):
        wid = lax.axis_index("s") * nc + lax.axis_index("c")
        base = wid * b_per_w
        pltpu.sync_copy(idx_hbm.at[pl.ds(base, b_per_w)], idx_v)
        cp = pltpu.async_copy(table_hbm.at[idx_v], rows_v, sem)
        zeros = jnp.zeros((nl,), jnp.float32)

        def zbody(i, carry):
            cnt_v[pl.ds(i * nl, nl)] = zeros
            return carry

        jax.lax.fori_loop(0, _K // nl, zbody, 0)
        ones = jnp.ones((nl,), jnp.float32)

        def hbody(i, carry):
            ii = idx_v[pl.ds(i * nl, nl)]
            plsc.addupdate_scatter(cnt_v, [ii], ones)
            return carry

        jax.lax.fori_loop(0, b_per_w // nl, hbody, 0)
        cp.wait()
        pltpu.sync_copy(rows_v, out_hbm.at[pl.ds(base, b_per_w)])
        pltpu.sync_copy(cnt_v, cnt_hbm.at[wid])

    return k(table, idx)


def kernel(inputs, embedding):
    flat = inputs.reshape(-1, _D)
    distances = (jnp.sum(flat ** 2, axis=1, keepdims=True)
                 + jnp.sum(embedding ** 2, axis=1)
                 - 2.0 * jnp.matmul(flat, embedding.T))
    idx = jnp.argmin(distances, axis=1)
    gathered, counts = _sc_gather_hist(embedding, idx)
    quantized, loss, perp = _tc_finalize(flat, gathered, counts)
    enc = jnp.zeros((flat.shape[0], _K), dtype=jnp.float32)
    enc = enc.at[jnp.arange(flat.shape[0]), idx].set(1.0)
    return (loss[0, 0], quantized.reshape(inputs.shape), perp[0, 0], enc)


# speedup vs baseline: 1.0220x; 1.0220x over previous
import functools

import jax
import jax.numpy as jnp
from jax import lax
from jax.experimental import pallas as pl
from jax.experimental.pallas import tpu as pltpu
from jax.experimental.pallas import tpu_sc as plsc

_K = 8192
_D = 32
_ROWS = 128
_COMMIT = 0.25


def _finalize_body(z_ref, g_ref, counts_ref, q_ref, loss_ref, perp_ref):
    z = z_ref[...]
    g = g_ref[...]
    q_ref[...] = z + (g - z)
    diff = g - z
    n_elems = z.shape[0] * _D
    loss_ref[...] = jnp.reshape(
        (1.0 + _COMMIT) * jnp.sum(diff * diff) / n_elems, (1, 1))
    counts = jnp.sum(counts_ref[...], axis=0, keepdims=True)
    p = counts * (1.0 / z.shape[0])
    perp_ref[...] = jnp.reshape(
        jnp.exp(-jnp.sum(p * jnp.log(p + 1e-10))), (1, 1))


def _tc_finalize(flat, gathered, counts):
    n = flat.shape[0]
    return pl.pallas_call(
        _finalize_body,
        out_shape=[
            jax.ShapeDtypeStruct((n, _D), jnp.float32),
            jax.ShapeDtypeStruct((1, 1), jnp.float32),
            jax.ShapeDtypeStruct((1, 1), jnp.float32),
        ],
    )(flat, gathered, counts)


def _sc_gather_hist(table, idx):
    info = plsc.get_sparse_core_info()
    nc, nl = info.num_cores, info.num_lanes
    nw = nc * info.num_subcores
    b, d = idx.shape[0], table.shape[1]
    b_per_w = b // nw
    mesh = plsc.VectorSubcoreMesh(core_axis_name="c", subcore_axis_name="s")

    @functools.partial(
        pl.kernel, mesh=mesh,
        out_type=[
            jax.ShapeDtypeStruct((b, d), jnp.float32),
            jax.ShapeDtypeStruct((nw, _K), jnp.float32),
        ],
        scratch_types=[
            pltpu.VMEM((b_per_w,), jnp.int32),
            pltpu.VMEM((b_per_w, d), jnp.float32),
            pltpu.VMEM((_K,), jnp.float32),
            pltpu.SemaphoreType.DMA,
        ],
        compiler_params=pltpu.CompilerParams(
            use_tc_tiling_on_sc=False, needs_layout_passes=False),
    )
    def k(table_hbm, idx_hbm, out_hbm, cnt_hbm, idx_v, rows_v, cnt_v, sem):
        wid = lax.axis_index("s") * nc + lax.axis_index("c")
        base = wid * b_per_w
        pltpu.sync_copy(idx_hbm.at[pl.ds(base, b_per_w)], idx_v)
        cp = pltpu.async_copy(table_hbm.at[idx_v], rows_v, sem)
        zeros = jnp.zeros((nl,), jnp.float32)

        def zbody(i, carry):
            cnt_v[pl.ds(i * nl, nl)] = zeros
            return carry

        jax.lax.fori_loop(0, _K // nl, zbody, 0)
        ones = jnp.ones((nl,), jnp.float32)

        def hbody(i, carry):
            ii = idx_v[pl.ds(i * nl, nl)]
            plsc.addupdate_scatter(cnt_v, [ii], ones)
            return carry

        jax.lax.fori_loop(0, b_per_w // nl, hbody, 0)
        cp.wait()
        pltpu.sync_copy(rows_v, out_hbm.at[pl.ds(base, b_per_w)])
        pltpu.sync_copy(cnt_v, cnt_hbm.at[wid])

    return k(table, idx)


def kernel(inputs, embedding):
    flat = inputs.reshape(-1, _D)
    distances = (jnp.sum(flat ** 2, axis=1, keepdims=True)
                 + jnp.sum(embedding ** 2, axis=1)
                 - 2.0 * jnp.matmul(flat, embedding.T))
    idx = jnp.argmin(distances, axis=1)
    gathered, counts = _sc_gather_hist(embedding, idx)
    quantized, loss, perp = _tc_finalize(flat, gathered, counts)
    enc = jnp.zeros((flat.shape[0], _K), dtype=jnp.float32)
    enc = enc.at[jnp.arange(flat.shape[0]), idx].set(1.0)
    return (loss[0, 0], quantized.reshape(inputs.shape), perp[0, 0], enc)
